# baseline (speedup 1.0000x reference)
import functools

import numpy as np

import jax
import jax.numpy as jnp
from jax.experimental import pallas as pl
from jax.experimental.pallas import tpu as pltpu

_N_CLASSES = 80
_A = 3
_CH = 5 + _N_CLASSES
_NCH = _A * _CH

_LANE = np.arange(_NCH) % _CH
_MASK_WH = (( _LANE == 2) | (_LANE == 3)).astype(np.float32)[None, None, :]
_MASK_GX = (_LANE == 0).astype(np.float32)[None, None, :]
_MASK_GY = (_LANE == 1).astype(np.float32)[None, None, :]
_IS_XY = (_LANE <= 1)


def _decode_block(x_ref, sc_ref, xy_ref, wh_ref, conf_ref, probs_ref, *, gw):
    blk = x_ref.shape[0]
    x = x_ref[...]
    lane = jax.lax.broadcasted_iota(jnp.int32, (1, 1, _NCH), 2).astype(jnp.float32)
    c = lane - _CH * jnp.floor(lane * (1.0 / _CH))
    en = jnp.exp(-x)
    v = 1.0 / jnp.where((c == 2.0) | (c == 3.0), en, 1.0 + en)
    j = pl.program_id(0)
    r = jax.lax.broadcasted_iota(jnp.int32, (blk, 1, 1), 0) + j * blk
    rf = r.astype(jnp.float32)
    gy = jnp.floor(rf * (1.0 / gw))
    gx = rf - float(gw) * gy
    g2 = jnp.concatenate([gx, gy], axis=2)
    inv_g = 1.0 / gw
    xy_p, wh_p, conf_p = [], [], []
    for a in range(_A):
        o = a * _CH
        probs_ref[:, a] = v[:, :, o + 5:o + _CH]
        xy_p.append((v[:, :, o:o + 2] + g2) * inv_g)
        wh_p.append(v[:, :, o + 2:o + 4] * sc_ref[:, :, o + 2:o + 4])
        conf_p.append(v[:, :, o + 4:o + 5])
    xy_ref[...] = jnp.concatenate(xy_p, axis=2).reshape(blk, -1)
    wh_ref[...] = jnp.concatenate(wh_p, axis=2).reshape(blk, -1)
    conf_ref[...] = jnp.concatenate(conf_p, axis=2).reshape(blk, -1)


@jax.jit
def kernel(feature_maps, input_shape, anchors):
    B, CHW, gh, gw = feature_maps.shape
    n = gh * gw
    xt = jnp.transpose(feature_maps, (2, 3, 0, 1)).reshape(n, B, CHW)

    ws = (anchors / input_shape[None, :]).reshape(_A, 2)
    base = np.ones((_NCH,), np.float32)
    base[_LANE == 0] = 0.0
    base[_LANE == 1] = 0.0
    xy_scale = np.zeros((_NCH,), np.float32)
    xy_scale[_LANE == 0] = 1.0 / gh
    xy_scale[_LANE == 1] = 1.0 / gw
    wh_lanes = np.where((_LANE == 2) | (_LANE == 3))[0]
    sc = jnp.asarray(base).at[wh_lanes].mul(ws.reshape(-1)) + jnp.asarray(xy_scale)
    sc3 = sc.reshape(1, 1, _NCH)

    blk = 152
    xy, wh, conf, probs = pl.pallas_call(
        functools.partial(_decode_block, gw=gw),
        grid=(n // blk,),
        in_specs=[
            pl.BlockSpec((blk, B, CHW), lambda j: (j, 0, 0)),
            pl.BlockSpec((1, 1, _NCH), lambda j: (0, 0, 0)),
        ],
        out_specs=[
            pl.BlockSpec((blk, B * 2 * _A), lambda j: (j, 0)),
            pl.BlockSpec((blk, B * 2 * _A), lambda j: (j, 0)),
            pl.BlockSpec((blk, B * _A), lambda j: (j, 0)),
            pl.BlockSpec((blk, _A, B, _N_CLASSES), lambda j: (j, 0, 0, 0)),
        ],
        out_shape=(
            jax.ShapeDtypeStruct((n, B * 2 * _A), jnp.float32),
            jax.ShapeDtypeStruct((n, B * 2 * _A), jnp.float32),
            jax.ShapeDtypeStruct((n, B * _A), jnp.float32),
            jax.ShapeDtypeStruct((n, _A, B, _N_CLASSES), jnp.float32),
        ),
        compiler_params=pltpu.CompilerParams(
            dimension_semantics=("parallel",),
        ),
    )(xt, sc3)

    box_probs = probs.reshape(gh, gw, _A, B, _N_CLASSES).transpose(3, 0, 1, 2, 4)
    box_xy = xy.reshape(gh, gw, B, _A, 2).transpose(2, 0, 1, 3, 4)
    box_wh = wh.reshape(gh, gw, B, _A, 2).transpose(2, 0, 1, 3, 4)
    box_conf = conf.reshape(gh, gw, B, _A, 1).transpose(2, 0, 1, 3, 4)
    return (box_xy, box_wh, box_conf, box_probs)


# speedup vs baseline: 1.0345x; 1.0345x over previous
import functools

import numpy as np

import jax
import jax.numpy as jnp
from jax.experimental import pallas as pl
from jax.experimental.pallas import tpu as pltpu

_N_CLASSES = 80
_A = 3
_CH = 5 + _N_CLASSES
_NCH = _A * _CH

_LANE = np.arange(_NCH) % _CH


def _decode_block(x_ref, sc_ref, xy_ref, wh_ref, conf_ref, probs_ref, *, gw):
    blk = x_ref.shape[0]
    x = x_ref[...]
    lane = jax.lax.broadcasted_iota(jnp.int32, (1, 1, _NCH), 2).astype(jnp.float32)
    c = lane - _CH * jnp.floor(lane * (1.0 / _CH))
    en = jnp.exp(-x)
    v = 1.0 / jnp.where((c == 2.0) | (c == 3.0), en, 1.0 + en)
    j = pl.program_id(0)
    r = jax.lax.broadcasted_iota(jnp.int32, (blk, 1, 1), 0) + j * blk
    rf = r.astype(jnp.float32)
    gy = jnp.floor(rf * (1.0 / gw))
    gx = rf - float(gw) * gy
    g2 = jnp.concatenate([gx, gy], axis=2)
    inv_g = 1.0 / gw
    xy_p, wh_p, conf_p = [], [], []
    for a in range(_A):
        o = a * _CH
        probs_ref[:, a] = v[:, :, o + 5:o + _CH]
        xy_p.append((v[:, :, o:o + 2] + g2) * inv_g)
        wh_p.append(v[:, :, o + 2:o + 4] * sc_ref[:, :, o + 2:o + 4])
        conf_p.append(v[:, :, o + 4:o + 5])
    xy_ref[...] = jnp.concatenate(xy_p, axis=2).reshape(blk, -1)
    wh_ref[...] = jnp.concatenate(wh_p, axis=2).reshape(blk, -1)
    conf_ref[...] = jnp.concatenate(conf_p, axis=2).reshape(blk, -1)


@jax.jit
def kernel(feature_maps, input_shape, anchors):
    B, CHW, gh, gw = feature_maps.shape
    n = gh * gw
    xt = jnp.transpose(feature_maps, (2, 3, 0, 1)).reshape(n, B, CHW)

    ws = (anchors / input_shape[None, :]).reshape(_A, 2)
    base = np.ones((_NCH,), np.float32)
    base[_LANE == 0] = 0.0
    base[_LANE == 1] = 0.0
    xy_scale = np.zeros((_NCH,), np.float32)
    xy_scale[_LANE == 0] = 1.0 / gh
    xy_scale[_LANE == 1] = 1.0 / gw
    wh_lanes = np.where((_LANE == 2) | (_LANE == 3))[0]
    sc = jnp.asarray(base).at[wh_lanes].mul(ws.reshape(-1)) + jnp.asarray(xy_scale)
    sc3 = sc.reshape(1, 1, _NCH)

    blk = 304
    xy, wh, conf, probs = pl.pallas_call(
        functools.partial(_decode_block, gw=gw),
        grid=(n // blk,),
        in_specs=[
            pl.BlockSpec((blk, B, CHW), lambda j: (j, 0, 0)),
            pl.BlockSpec((1, 1, _NCH), lambda j: (0, 0, 0)),
        ],
        out_specs=[
            pl.BlockSpec((blk, B * 2 * _A), lambda j: (j, 0)),
            pl.BlockSpec((blk, B * 2 * _A), lambda j: (j, 0)),
            pl.BlockSpec((blk, B * _A), lambda j: (j, 0)),
            pl.BlockSpec((blk, _A, B, _N_CLASSES), lambda j: (j, 0, 0, 0)),
        ],
        out_shape=(
            jax.ShapeDtypeStruct((n, B * 2 * _A), jnp.float32),
            jax.ShapeDtypeStruct((n, B * 2 * _A), jnp.float32),
            jax.ShapeDtypeStruct((n, B * _A), jnp.float32),
            jax.ShapeDtypeStruct((n, _A, B, _N_CLASSES), jnp.float32),
        ),
        compiler_params=pltpu.CompilerParams(
            dimension_semantics=("parallel",),
        ),
    )(xt, sc3)

    box_probs = probs.reshape(gh, gw, _A, B, _N_CLASSES).transpose(3, 0, 1, 2, 4)
    box_xy = xy.reshape(gh, gw, B, _A, 2).transpose(2, 0, 1, 3, 4)
    box_wh = wh.reshape(gh, gw, B, _A, 2).transpose(2, 0, 1, 3, 4)
    box_conf = conf.reshape(gh, gw, B, _A, 1).transpose(2, 0, 1, 3, 4)
    return (box_xy, box_wh, box_conf, box_probs)
